# Initial kernel scaffold; baseline (speedup 1.0000x reference)
import jax
import jax.numpy as jnp
from jax.experimental import pallas as pl


def kernel(x, edge_index, W, b):
    raise NotImplementedError("write your pallas kernel here")


# speedup vs baseline: 13.5497x; 13.5497x over previous
import functools

import jax
import jax.numpy as jnp
from jax import lax
from jax.experimental import pallas as pl
from jax.experimental.pallas import tpu as pltpu
from jax.experimental.pallas import tpu_sc as plsc

N = 10000
E = 320000
D = 128
K = 10
ALPHA = 0.1

NC, NS = 2, 16
NW = NC * NS
EPW = E // NW
CHUNK = 125
NCHUNK = EPW // CHUNK
NP = 10240
RPT = NP // NS
ZB = 128
DEGW = 128

_mesh = plsc.VectorSubcoreMesh(
    core_axis_name="c", subcore_axis_name="s", num_cores=NC, num_subcores=NS
)


def _zero_rows(buf, nrows, ncols):
    def body(i, _):
        for j in range(ncols // 16):
            buf[i, pl.ds(j * 16, 16)] = jnp.zeros((16,), jnp.float32)
        return 0

    lax.fori_loop(0, nrows, body, 0)


def _fill_ones(buf, nrows, ncols):
    def body(i, _):
        for j in range(ncols // 16):
            buf[i, pl.ds(j * 16, 16)] = jnp.ones((16,), jnp.float32)
        return 0

    lax.fori_loop(0, nrows, body, 0)


def _degree_body(coli_hbm, out_hbm, idx_c, buf, deg_sh):
    c = lax.axis_index("c")
    s = lax.axis_index("s")
    wid = c * NS + s
    base = s * RPT

    _zero_rows(buf, ZB, DEGW)
    for i in range(RPT // ZB):
        pltpu.sync_copy(buf, deg_sh.at[pl.ds(base + i * ZB, ZB)])
    _fill_ones(buf, ZB, DEGW)
    pltpu.sync_copy(coli_hbm.at[wid], idx_c)
    plsc.subcore_barrier()

    def step(j, _):
        pltpu.sync_copy(buf.at[pl.ds(0, CHUNK)], deg_sh.at[idx_c.at[j]], add=True)
        return 0

    lax.fori_loop(0, NCHUNK, step, 0)
    plsc.subcore_barrier()
    pltpu.sync_copy(deg_sh.at[pl.ds(base, RPT)], out_hbm.at[c, pl.ds(base, RPT)])


_degree = pl.kernel(
    _degree_body,
    out_type=jax.ShapeDtypeStruct((NC, NP, DEGW), jnp.float32),
    mesh=_mesh,
    scratch_types=[
        pltpu.VMEM((NCHUNK, CHUNK), jnp.int32),
        pltpu.VMEM((ZB, DEGW), jnp.float32),
        pltpu.VMEM_SHARED((NP, DEGW), jnp.float32),
    ],
)


def _propagate_body(g_hbm, rowi_hbm, coli_hbm, out_hbm, idx_r, idx_c, buf, s_sh):
    c = lax.axis_index("c")
    s = lax.axis_index("s")
    wid = c * NS + s
    base = s * RPT

    _zero_rows(buf, ZB, D)
    for i in range(RPT // ZB):
        pltpu.sync_copy(buf, s_sh.at[pl.ds(base + i * ZB, ZB)])
    pltpu.sync_copy(rowi_hbm.at[wid], idx_r)
    pltpu.sync_copy(coli_hbm.at[wid], idx_c)
    plsc.subcore_barrier()

    def step(j, _):
        pltpu.sync_copy(g_hbm.at[idx_r.at[j]], buf.at[pl.ds(0, CHUNK)])
        pltpu.sync_copy(buf.at[pl.ds(0, CHUNK)], s_sh.at[idx_c.at[j]], add=True)
        return 0

    lax.fori_loop(0, NCHUNK, step, 0)
    plsc.subcore_barrier()
    pltpu.sync_copy(s_sh.at[pl.ds(base, RPT)], out_hbm.at[c, pl.ds(base, RPT)])


_propagate = pl.kernel(
    _propagate_body,
    out_type=jax.ShapeDtypeStruct((NC, NP, D), jnp.float32),
    mesh=_mesh,
    scratch_types=[
        pltpu.VMEM((NCHUNK, CHUNK), jnp.int32),
        pltpu.VMEM((NCHUNK, CHUNK), jnp.int32),
        pltpu.VMEM((ZB, D), jnp.float32),
        pltpu.VMEM_SHARED((NP, D), jnp.float32),
    ],
)


_BLK = 400


def _linear_body(x_ref, w_ref, b_ref, o_ref):
    o_ref[...] = (
        lax.dot_general(
            x_ref[...], w_ref[...], (((1,), (1,)), ((), ())),
            preferred_element_type=jnp.float32,
            precision=lax.Precision.HIGHEST,
        )
        + b_ref[...]
    )


def _linear(x, W, b2):
    return pl.pallas_call(
        _linear_body,
        out_shape=jax.ShapeDtypeStruct((N, D), jnp.float32),
        grid=(N // _BLK,),
        in_specs=[
            pl.BlockSpec((_BLK, D), lambda i: (i, 0)),
            pl.BlockSpec((D, D), lambda i: (0, 0)),
            pl.BlockSpec((1, D), lambda i: (0, 0)),
        ],
        out_specs=pl.BlockSpec((_BLK, D), lambda i: (i, 0)),
    )(x, W, b2)


def _prep_body(d_ref, h_ref, u_ref, s2_ref, g0_ref):
    deg = d_ref[0] + d_ref[1] + 1.0
    dinv = jnp.where(deg > 0, lax.rsqrt(jnp.maximum(deg, 1e-12)), 0.0)
    col = dinv[:, 0:1]
    u_ref[...] = (1.0 - ALPHA) * col * col
    s2_ref[...] = (1.0 - ALPHA) * col
    g0_ref[...] = h_ref[...] * col


def _prep(Dp, h0):
    return pl.pallas_call(
        _prep_body,
        out_shape=(
            jax.ShapeDtypeStruct((N, 1), jnp.float32),
            jax.ShapeDtypeStruct((N, 1), jnp.float32),
            jax.ShapeDtypeStruct((N, D), jnp.float32),
        ),
        grid=(N // _BLK,),
        in_specs=[
            pl.BlockSpec((NC, _BLK, DEGW), lambda i: (0, i, 0)),
            pl.BlockSpec((_BLK, D), lambda i: (i, 0)),
        ],
        out_specs=(
            pl.BlockSpec((_BLK, 1), lambda i: (i, 0)),
            pl.BlockSpec((_BLK, 1), lambda i: (i, 0)),
            pl.BlockSpec((_BLK, D), lambda i: (i, 0)),
        ),
    )(Dp, h0)


def _combine_body(p_ref, g_ref, b_ref, sc_ref, o_ref):
    t = p_ref[0] + p_ref[1] + g_ref[...]
    o_ref[...] = sc_ref[...] * t + ALPHA * b_ref[...]


def _combine(P, g, base, scale):
    return pl.pallas_call(
        _combine_body,
        out_shape=jax.ShapeDtypeStruct((N, D), jnp.float32),
        grid=(N // _BLK,),
        in_specs=[
            pl.BlockSpec((NC, _BLK, D), lambda i: (0, i, 0)),
            pl.BlockSpec((_BLK, D), lambda i: (i, 0)),
            pl.BlockSpec((_BLK, D), lambda i: (i, 0)),
            pl.BlockSpec((_BLK, 1), lambda i: (i, 0)),
        ],
        out_specs=pl.BlockSpec((_BLK, D), lambda i: (i, 0)),
    )(P, g, base, scale)


def kernel(x, edge_index, W, b):
    ei = edge_index.astype(jnp.int32)
    rowi = ei[0].reshape(NW, NCHUNK, CHUNK)
    coli = ei[1].reshape(NW, NCHUNK, CHUNK)

    h0 = _linear(x, W, b.reshape(1, D))
    Dp = _degree(coli)
    u, s2, g0 = _prep(Dp, h0)

    g = g0
    for k in range(K):
        P = _propagate(g, rowi, coli)
        if k < K - 1:
            g = _combine(P, g, g0, u)
        else:
            h = _combine(P, g, h0, s2)
    return h
